# baseline (speedup 1.0000x reference)
import functools

import jax
import jax.numpy as jnp
from jax import lax
from jax.experimental import pallas as pl
from jax.experimental.pallas import tpu as pltpu
from jax.experimental.pallas import tpu_sc as plsc

_INTERPRET = False

EB = 512
NB = 1000
NW = 32
CH = 40


def _sc_mesh():
    return plsc.VectorSubcoreMesh(core_axis_name="c", subcore_axis_name="s")


def _sc_gather(tab, idx):
    e = idx.shape[0]
    cols = tab.shape[1]
    per_w = e // NW
    nch = per_w // CH

    @functools.partial(
        pl.kernel, mesh=_sc_mesh(),
        out_type=jax.ShapeDtypeStruct((e, cols), jnp.float32),
        scratch_types=[
            pltpu.VMEM((per_w,), jnp.int32),
            pltpu.VMEM((CH, cols), jnp.float32),
            pltpu.SemaphoreType.DMA,
        ],
    )
    def k(tab_hbm, idx_hbm, out_hbm, idx_v, rows_v, sem):
        wid = lax.axis_index("c") * 16 + lax.axis_index("s")
        base = wid * per_w
        pltpu.sync_copy(idx_hbm.at[pl.ds(base, per_w)], idx_v)

        def body(j, _):
            off = pl.multiple_of(j * CH, CH)
            pltpu.async_copy(tab_hbm.at[idx_v.at[pl.ds(off, CH)]],
                             rows_v, sem).wait()
            pltpu.sync_copy(rows_v, out_hbm.at[pl.ds(base + off, CH)])
            return 0

        lax.fori_loop(0, nch, body, 0)

    return k(tab, idx)


def _sc_aggregate(expaw, av, dst4d, n):
    e = av.shape[0]
    per_t = e // 16
    nch = per_t // CH
    nchp = dst4d.shape[2]
    npad = ((n + 2047) // 2048) * 2048
    half = npad // 2
    rpt = half // 16
    qr = rpt // 10
    nq = rpt // qr

    zeros = jnp.zeros((rpt, 128), jnp.float32)

    @functools.partial(
        pl.kernel, mesh=_sc_mesh(),
        out_type=jax.ShapeDtypeStruct((5, npad, 128), jnp.float32),
        scratch_types=[
            pltpu.VMEM((nchp, CH), jnp.int32),
            pltpu.VMEM((CH, 16), jnp.float32),
            pltpu.VMEM((CH, 128), jnp.float32),
            pltpu.VMEM((CH, 128), jnp.float32),
            pltpu.VMEM((qr, 128), jnp.float32),
            pltpu.VMEM_SHARED((half + 8, 128), jnp.float32),
        ],
    )
    def k(ea_hbm, av_hbm, dst_hbm, zeros_hbm, out_hbm,
          idx_v, ea_v, av_v, vals_v, stage_v, table):
        sc = lax.axis_index("c")
        tid = lax.axis_index("s")
        base = pl.multiple_of(tid * per_t, 8)
        row0 = pl.multiple_of(tid * rpt, 8)
        orow0 = pl.multiple_of(sc * half + tid * rpt, 8)
        pltpu.sync_copy(dst_hbm.at[sc, tid], idx_v)

        for h in range(5):
            pltpu.sync_copy(zeros_hbm, table.at[pl.ds(row0, rpt)])
            if h == 4:
                def zv(ei, _):
                    for c in range(1, 8):
                        vals_v[ei, pl.ds(c * 16, 16)] = jnp.zeros(
                            (16,), jnp.float32)
                    return 0
                lax.fori_loop(0, CH, zv, 0)
            plsc.subcore_barrier()

            def chunk(j, _):
                eoff = pl.multiple_of(base + j * CH, 8)
                pltpu.sync_copy(ea_hbm.at[pl.ds(eoff, CH)], ea_v)
                if h < 4:
                    pltpu.sync_copy(av_hbm.at[pl.ds(eoff, CH)], av_v)

                    def body(ei, _):
                        evec = ea_v[ei, pl.ds(0, 16)]
                        scv = evec[h]
                        for c in range(8):
                            vals_v[ei, pl.ds(c * 16, 16)] = (
                                scv * av_v[ei, pl.ds(c * 16, 16)])
                        return 0
                else:
                    def body(ei, _):
                        vals_v[ei, pl.ds(0, 16)] = ea_v[ei, pl.ds(0, 16)]
                        return 0
                lax.fori_loop(0, CH, body, 0)
                pltpu.sync_copy(vals_v, table.at[idx_v.at[j]], add=True)
                return 0
            lax.fori_loop(0, nch, chunk, 0)
            plsc.subcore_barrier()

            for q in range(nq):
                pltpu.sync_copy(
                    table.at[pl.ds(pl.multiple_of(row0 + q * qr, 8), qr)],
                    stage_v)
                pltpu.sync_copy(
                    stage_v,
                    out_hbm.at[h, pl.ds(pl.multiple_of(orow0 + q * qr, 8),
                                        qr)])

    return k(expaw, av, dst4d, zeros)


def _gelu(x):
    return x * 0.5 * (1.0 + jax.lax.erf(x / jnp.sqrt(2.0).astype(x.dtype)))


def _node_pre_body(s0_ref, rtc_ref, rand_ref, seqWT_ref, wd0T_ref, b1a0_ref,
                   tab_ref, p0_ref):
    s0 = s0_ref[...]
    rand = rand_ref[...]
    r = jnp.dot(rtc_ref[...], seqWT_ref[...],
                preferred_element_type=jnp.float32)
    tab_ref[:, 0:128] = s0
    tab_ref[:, 128:256] = r
    tab_ref[:, 256:257] = rand
    tab_ref[:, 257:384] = jnp.zeros_like(tab_ref[:, 257:384])
    p0_ref[:, 0:128] = jnp.dot(s0, wd0T_ref[...],
                               preferred_element_type=jnp.float32) + b1a0_ref[...]
    p0_ref[:, 128:129] = rand
    p0_ref[:, 129:256] = jnp.zeros_like(p0_ref[:, 129:256])


def _node_pre(s0, rtc, rand, seqWT, wd0T, b1a0):
    n = s0.shape[0]
    grid = (n // NB,)
    tab, p0 = pl.pallas_call(
        _node_pre_body,
        grid=grid,
        in_specs=[
            pl.BlockSpec((NB, 128), lambda i: (i, 0)),
            pl.BlockSpec((NB, 33), lambda i: (i, 0)),
            pl.BlockSpec((NB, 1), lambda i: (i, 0)),
            pl.BlockSpec((33, 128), lambda i: (0, 0)),
            pl.BlockSpec((128, 128), lambda i: (0, 0)),
            pl.BlockSpec((1, 128), lambda i: (0, 0)),
        ],
        out_specs=[
            pl.BlockSpec((NB, 384), lambda i: (i, 0)),
            pl.BlockSpec((NB, 256), lambda i: (i, 0)),
        ],
        out_shape=[
            jax.ShapeDtypeStruct((n, 384), jnp.float32),
            jax.ShapeDtypeStruct((n, 256), jnp.float32),
        ],
        interpret=_INTERPRET,
    )(s0, rtc, rand[:, None], seqWT, wd0T, b1a0)
    return tab, p0


def _edge_body(first, z_ref, srg_ref, pd_ref,
               waT_ref, w2aT_ref, b2a_ref, w3aT_ref, b3a_ref,
               wvT_ref, b1v_ref, w2vT_ref, b2v_ref, w3vT_ref, b3v_ref,
               seqb_ref,
               expaw_ref, av_ref, u_ref):
    z = z_ref[...]
    if first:
        vis = jnp.where(srg_ref[:, 256:257] < pd_ref[:, 128:129], 1.0, 0.0)
        u = (srg_ref[:, 0:128]
             + vis * srg_ref[:, 128:256]
             + seqb_ref[...])
        u_ref[...] = u
    else:
        u = srg_ref[...]
    zu = jnp.concatenate([z, u], axis=1)
    h = (jnp.dot(zu, waT_ref[...], preferred_element_type=jnp.float32)
         + pd_ref[:, 0:128])
    h = _gelu(h)
    h = _gelu(jnp.dot(h, w2aT_ref[...], preferred_element_type=jnp.float32)
              + b2a_ref[...])
    aw = jnp.dot(h, w3aT_ref[...], preferred_element_type=jnp.float32) + b3a_ref[...]
    expaw_ref[:, 0:4] = jnp.exp(aw)
    expaw_ref[:, 4:16] = jnp.zeros_like(expaw_ref[:, 4:16])
    g = _gelu(jnp.dot(zu, wvT_ref[...], preferred_element_type=jnp.float32)
              + b1v_ref[...])
    g = _gelu(jnp.dot(g, w2vT_ref[...], preferred_element_type=jnp.float32)
              + b2v_ref[...])
    av_ref[...] = (jnp.dot(g, w3vT_ref[...], preferred_element_type=jnp.float32)
                   + b3v_ref[...])


def _edge_kernel(first, z, srg, pd, lw, seqb):
    e = z.shape[0]
    grid = (e // EB,)
    (waT, w2aT, b2a, w3aT, b3a, wvT, b1v, w2vT, b2v, w3vT, b3v) = lw
    full = lambda shape: pl.BlockSpec(shape, lambda i: tuple(0 for _ in shape))
    srg_cols = srg.shape[1]
    pd_cols = pd.shape[1]
    outs = pl.pallas_call(
        functools.partial(_edge_body, first),
        grid=grid,
        in_specs=[
            pl.BlockSpec((EB, 128), lambda i: (i, 0)),
            pl.BlockSpec((EB, srg_cols), lambda i: (i, 0)),
            pl.BlockSpec((EB, pd_cols), lambda i: (i, 0)),
            full((256, 128)), full((128, 128)), full((1, 128)),
            full((128, 4)), full((1, 4)),
            full((256, 128)), full((1, 128)), full((128, 128)), full((1, 128)),
            full((128, 128)), full((1, 128)),
            full((1, 128)),
        ],
        out_specs=[
            pl.BlockSpec((EB, 16), lambda i: (i, 0)),
            pl.BlockSpec((EB, 128), lambda i: (i, 0)),
            pl.BlockSpec((EB, 128), lambda i: (i, 0)),
        ],
        out_shape=[
            jax.ShapeDtypeStruct((e, 16), jnp.float32),
            jax.ShapeDtypeStruct((e, 128), jnp.float32),
            jax.ShapeDtypeStruct((e, 128), jnp.float32),
        ],
        interpret=_INTERPRET,
    )(z, srg, pd, waT, w2aT, b2a, w3aT, b3a, wvT, b1v, w2vT, b2v,
      w3vT, b3v, seqb)
    return outs


def _update_body(last, num_ref, s_ref, woutT_ref, bout_ref,
                 wf1T_ref, bf1_ref, wf2T_ref, bf2_ref, wnT_ref, bn_ref,
                 s_out_ref, p_out_ref):
    inv = 1.0 / jnp.sqrt(1.0 + 1e-5)
    ao = jnp.concatenate(
        [num_ref[h, :, 0:128] / (num_ref[4, :, h:h + 1] + 1e-12)
         for h in range(4)], axis=1)
    s = s_ref[...]
    s = s + (jnp.dot(ao, woutT_ref[...], preferred_element_type=jnp.float32)
             + bout_ref[...]) * inv
    t = _gelu(jnp.dot(s, wf1T_ref[...], preferred_element_type=jnp.float32)
              + bf1_ref[...])
    s = s + (jnp.dot(t, wf2T_ref[...], preferred_element_type=jnp.float32)
             + bf2_ref[...]) * inv
    s_out_ref[...] = s
    p = jnp.dot(s, wnT_ref[...], preferred_element_type=jnp.float32) + bn_ref[...]
    p_out_ref[...] = p
    del last


def _update_kernel(num, s, woutT, bout, wf1T, bf1, wf2T, bf2, wnT, bn):
    n = s.shape[0]
    pc = wnT.shape[1]
    grid = (n // NB,)
    full = lambda shape: pl.BlockSpec(shape, lambda i: tuple(0 for _ in shape))
    s_new, p_new = pl.pallas_call(
        functools.partial(_update_body, False),
        grid=grid,
        in_specs=[
            pl.BlockSpec((5, NB, 128), lambda i: (0, i, 0)),
            pl.BlockSpec((NB, 128), lambda i: (i, 0)),
            full((512, 128)), full((1, 128)),
            full((128, 128)), full((1, 128)),
            full((128, 128)), full((1, 128)),
            full((128, pc)), full((1, pc)),
        ],
        out_specs=[
            pl.BlockSpec((NB, 128), lambda i: (i, 0)),
            pl.BlockSpec((NB, pc), lambda i: (i, 0)),
        ],
        out_shape=[
            jax.ShapeDtypeStruct((n, 128), jnp.float32),
            jax.ShapeDtypeStruct((n, pc), jnp.float32),
        ],
        interpret=_INTERPRET,
    )(num, s, woutT, bout, wf1T, bf1, wf2T, bf2, wnT, bn)
    return s_new, p_new


def _gather_rows(tab, idx):
    return tab[idx]


def _vis_compute(rand, src, dst):
    return (rand[src] < rand[dst]).astype(jnp.float32)


def _aggregate(expaw, av, dst, n):
    e = expaw.shape[0]
    half = e // 2
    out = []
    for p in range(2):
        sl = slice(p * half, (p + 1) * half)
        vals = jnp.concatenate(
            [av[sl], jnp.ones((half, 1), jnp.float32),
             jnp.zeros((half, 15), jnp.float32)], axis=1)
        per_h = []
        for h in range(4):
            per_h.append(jax.ops.segment_sum(
                expaw[sl, h:h + 1] * vals, dst[sl], num_segments=n))
        out.append(jnp.stack(per_h, axis=0))
    return jnp.stack(out, axis=0)


def kernel(s, z, edge_idx, valid_mask, res_type_clone, params):
    n, d = s.shape
    kk = res_type_clone.shape[-1]
    src = edge_idx[0].astype(jnp.int32)
    dst = edge_idx[1].astype(jnp.int32)
    rand = jax.random.uniform(jax.random.key(42), (n,), dtype=s.dtype)
    rtc = (res_type_clone != 0).reshape(-1, kk).astype(s.dtype)

    seqW, seqb = params["seq_to_s"]
    layers = params["layers"]

    def lt(p):
        W, b = p
        return W.T, b[None, :]

    lws = []
    for lp in layers:
        w1a, b1a = lp["aw"][0]
        w2aT, b2a = lt(lp["aw"][1])
        w3aT, b3a = lt(lp["aw"][2])
        wvT, b1v = lt(lp["av"][0])
        w2vT, b2v = lt(lp["av"][1])
        w3vT, b3v = lt(lp["av"][2])
        wdT = w1a[:, 0:128].T
        waT = w1a[:, 128:384].T
        lws.append(dict(wdT=wdT, b1a=b1a[None, :],
                        ew=(waT, w2aT, b2a, w3aT, b3a,
                            wvT, b1v, w2vT, b2v, w3vT, b3v)))

    tab, p = _node_pre(s, rtc, rand, seqW.T, lws[0]["wdT"], lws[0]["b1a"])

    srg0 = _sc_gather(tab, src)
    npad = ((n + 2047) // 2048) * 2048
    half = npad // 2
    nch = z.shape[0] // 16 // CH
    nchp = ((nch + 7) // 8) * 8
    d0 = jnp.where(dst < half, dst, half)
    d1 = jnp.where(dst >= half, dst - half, half)
    dst4d = jnp.pad(
        jnp.stack([d0, d1]).reshape(2, 16, nch, CH),
        ((0, 0), (0, 0), (0, nchp - nch), (0, 0)))

    u = None
    cur_s = s
    for li, lp in enumerate(layers):
        pd = _sc_gather(p, dst)
        if li == 0:
            expaw, av, u = _edge_kernel(True, z, srg0, pd,
                                        lws[li]["ew"], seqb[None, :])
        else:
            expaw, av, _ = _edge_kernel(False, z, u, pd,
                                        lws[li]["ew"], seqb[None, :])
        num = _sc_aggregate(expaw, av, dst4d, n)
        woutT, bout = lt(layers[li]["out"])
        wf1T, bf1 = lt(layers[li]["ffn"][0])
        wf2T, bf2 = lt(layers[li]["ffn"][1])
        if li + 1 < len(layers):
            wnT, bn = lws[li + 1]["wdT"], lws[li + 1]["b1a"]
        else:
            predW = params["pred_W"]
            wnT, bn = predW.T, jnp.zeros((1, predW.shape[0]), jnp.float32)
        cur_s, p = _update_kernel(num, cur_s, woutT, bout, wf1T, bf1,
                                  wf2T, bf2, wnT, bn)

    logits = p
    bm, nn = valid_mask.shape
    return logits.reshape(bm, nn, kk)


# speedup vs baseline: 2.0830x; 2.0830x over previous
import functools

import jax
import jax.numpy as jnp
from jax import lax
from jax.experimental import pallas as pl
from jax.experimental.pallas import tpu as pltpu
from jax.experimental.pallas import tpu_sc as plsc

_INTERPRET = False

EB = 512
NB = 1000
NW = 32
CH = 80


def _sc_mesh():
    return plsc.VectorSubcoreMesh(core_axis_name="c", subcore_axis_name="s")


def _sc_gather(tab, idx):
    e = idx.shape[0]
    cols = tab.shape[1]
    per_w = e // NW
    nch = per_w // CH

    @functools.partial(
        pl.kernel, mesh=_sc_mesh(),
        out_type=jax.ShapeDtypeStruct((e, cols), jnp.float32),
        scratch_types=[
            pltpu.VMEM((per_w,), jnp.int32),
            pltpu.VMEM((CH, cols), jnp.float32),
            pltpu.SemaphoreType.DMA,
        ],
    )
    def k(tab_hbm, idx_hbm, out_hbm, idx_v, rows_v, sem):
        wid = lax.axis_index("c") * 16 + lax.axis_index("s")
        base = wid * per_w
        pltpu.sync_copy(idx_hbm.at[pl.ds(base, per_w)], idx_v)

        def body(j, _):
            off = pl.multiple_of(j * CH, CH)
            pltpu.async_copy(tab_hbm.at[idx_v.at[pl.ds(off, CH)]],
                             rows_v, sem).wait()
            pltpu.sync_copy(rows_v, out_hbm.at[pl.ds(base + off, CH)])
            return 0

        lax.fori_loop(0, nch, body, 0)

    return k(tab, idx)


def _sc_aggregate(expaw, av, dst3d, n):
    e = av.shape[0]
    per_w = e // NW
    nch = per_w // CH
    nchp = dst3d.shape[1]
    npad = ((n + 2047) // 2048) * 2048
    rpt = npad // 16
    qr = CH
    nq = rpt // qr

    @functools.partial(
        pl.kernel, mesh=_sc_mesh(),
        out_type=jax.ShapeDtypeStruct((2, 5, npad, 128), jnp.float32),
        scratch_types=[
            pltpu.VMEM((nchp, CH), jnp.int32),
            pltpu.VMEM((CH, 16), jnp.float32),
            pltpu.VMEM((CH, 128), jnp.float32),
            pltpu.VMEM((CH, 128), jnp.float32),
            pltpu.VMEM_SHARED((npad, 128), jnp.float32),
        ],
    )
    def k(ea_hbm, av_hbm, dst_hbm, out_hbm,
          idx_v, ea_v, av_v, vals_v, table):
        sc = lax.axis_index("c")
        tid = lax.axis_index("s")
        wid = sc * 16 + tid
        base = pl.multiple_of(wid * per_w, 8)
        row0 = pl.multiple_of(tid * rpt, 8)
        pltpu.sync_copy(dst_hbm.at[wid], idx_v)

        for h in range(5):
            def zrow(q, _):
                for c in range(8):
                    av_v[q, pl.ds(c * 16, 16)] = jnp.zeros(
                        (16,), jnp.float32)
                return 0
            lax.fori_loop(0, qr, zrow, 0)
            for q in range(nq):
                pltpu.sync_copy(
                    av_v,
                    table.at[pl.ds(pl.multiple_of(row0 + q * qr, 8), qr)])
            if h == 4:
                def zv(ei, _):
                    for c in range(1, 8):
                        vals_v[ei, pl.ds(c * 16, 16)] = jnp.zeros(
                            (16,), jnp.float32)
                    return 0
                lax.fori_loop(0, CH, zv, 0)
            plsc.subcore_barrier()

            def chunk(j, _):
                eoff = pl.multiple_of(base + j * CH, 8)
                pltpu.sync_copy(ea_hbm.at[pl.ds(eoff, CH)], ea_v)
                if h < 4:
                    pltpu.sync_copy(av_hbm.at[pl.ds(eoff, CH)], av_v)

                    def body(ei, _):
                        evec = ea_v[ei, pl.ds(0, 16)]
                        scv = evec[h]
                        for c in range(8):
                            vals_v[ei, pl.ds(c * 16, 16)] = (
                                scv * av_v[ei, pl.ds(c * 16, 16)])
                        return 0
                else:
                    def body(ei, _):
                        vals_v[ei, pl.ds(0, 16)] = ea_v[ei, pl.ds(0, 16)]
                        return 0
                lax.fori_loop(0, CH, body, 0)
                pltpu.sync_copy(vals_v, table.at[idx_v.at[j]], add=True)
                return 0
            lax.fori_loop(0, nch, chunk, 0)
            plsc.subcore_barrier()

            for q in range(nq):
                roff = pl.multiple_of(row0 + q * qr, 8)
                pltpu.sync_copy(table.at[pl.ds(roff, qr)], vals_v)
                pltpu.sync_copy(vals_v, out_hbm.at[sc, h, pl.ds(roff, qr)])
            plsc.subcore_barrier()

    return k(expaw, av, dst3d)


def _gelu(x):
    return x * 0.5 * (1.0 + jax.lax.erf(x / jnp.sqrt(2.0).astype(x.dtype)))


def _node_pre_body(s0_ref, rtc_ref, rand_ref, seqWT_ref, wd0T_ref, b1a0_ref,
                   tab_ref, p0_ref):
    s0 = s0_ref[...]
    rand = rand_ref[...]
    r = jnp.dot(rtc_ref[...], seqWT_ref[...],
                preferred_element_type=jnp.float32)
    tab_ref[:, 0:128] = s0
    tab_ref[:, 128:256] = r
    tab_ref[:, 256:257] = rand
    tab_ref[:, 257:384] = jnp.zeros_like(tab_ref[:, 257:384])
    p0_ref[:, 0:128] = jnp.dot(s0, wd0T_ref[...],
                               preferred_element_type=jnp.float32) + b1a0_ref[...]
    p0_ref[:, 128:129] = rand
    p0_ref[:, 129:256] = jnp.zeros_like(p0_ref[:, 129:256])


def _node_pre(s0, rtc, rand, seqWT, wd0T, b1a0):
    n = s0.shape[0]
    grid = (n // NB,)
    tab, p0 = pl.pallas_call(
        _node_pre_body,
        grid=grid,
        in_specs=[
            pl.BlockSpec((NB, 128), lambda i: (i, 0)),
            pl.BlockSpec((NB, 33), lambda i: (i, 0)),
            pl.BlockSpec((NB, 1), lambda i: (i, 0)),
            pl.BlockSpec((33, 128), lambda i: (0, 0)),
            pl.BlockSpec((128, 128), lambda i: (0, 0)),
            pl.BlockSpec((1, 128), lambda i: (0, 0)),
        ],
        out_specs=[
            pl.BlockSpec((NB, 384), lambda i: (i, 0)),
            pl.BlockSpec((NB, 256), lambda i: (i, 0)),
        ],
        out_shape=[
            jax.ShapeDtypeStruct((n, 384), jnp.float32),
            jax.ShapeDtypeStruct((n, 256), jnp.float32),
        ],
        interpret=_INTERPRET,
    )(s0, rtc, rand[:, None], seqWT, wd0T, b1a0)
    return tab, p0


def _edge_body(first, z_ref, srg_ref, pd_ref,
               waT_ref, w2aT_ref, b2a_ref, w3aT_ref, b3a_ref,
               wvT_ref, b1v_ref, w2vT_ref, b2v_ref, w3vT_ref, b3v_ref,
               seqb_ref,
               expaw_ref, av_ref, u_ref):
    z = z_ref[...]
    if first:
        vis = jnp.where(srg_ref[:, 256:257] < pd_ref[:, 128:129], 1.0, 0.0)
        u = (srg_ref[:, 0:128]
             + vis * srg_ref[:, 128:256]
             + seqb_ref[...])
        u_ref[...] = u
    else:
        u = srg_ref[...]
    zu = jnp.concatenate([z, u], axis=1)
    h = (jnp.dot(zu, waT_ref[...], preferred_element_type=jnp.float32)
         + pd_ref[:, 0:128])
    h = _gelu(h)
    h = _gelu(jnp.dot(h, w2aT_ref[...], preferred_element_type=jnp.float32)
              + b2a_ref[...])
    aw = jnp.dot(h, w3aT_ref[...], preferred_element_type=jnp.float32) + b3a_ref[...]
    expaw_ref[:, 0:4] = jnp.exp(aw)
    expaw_ref[:, 4:16] = jnp.zeros_like(expaw_ref[:, 4:16])
    g = _gelu(jnp.dot(zu, wvT_ref[...], preferred_element_type=jnp.float32)
              + b1v_ref[...])
    g = _gelu(jnp.dot(g, w2vT_ref[...], preferred_element_type=jnp.float32)
              + b2v_ref[...])
    av_ref[...] = (jnp.dot(g, w3vT_ref[...], preferred_element_type=jnp.float32)
                   + b3v_ref[...])


def _edge_kernel(first, z, srg, pd, lw, seqb):
    e = z.shape[0]
    grid = (e // EB,)
    (waT, w2aT, b2a, w3aT, b3a, wvT, b1v, w2vT, b2v, w3vT, b3v) = lw
    full = lambda shape: pl.BlockSpec(shape, lambda i: tuple(0 for _ in shape))
    srg_cols = srg.shape[1]
    pd_cols = pd.shape[1]
    outs = pl.pallas_call(
        functools.partial(_edge_body, first),
        grid=grid,
        in_specs=[
            pl.BlockSpec((EB, 128), lambda i: (i, 0)),
            pl.BlockSpec((EB, srg_cols), lambda i: (i, 0)),
            pl.BlockSpec((EB, pd_cols), lambda i: (i, 0)),
            full((256, 128)), full((128, 128)), full((1, 128)),
            full((128, 4)), full((1, 4)),
            full((256, 128)), full((1, 128)), full((128, 128)), full((1, 128)),
            full((128, 128)), full((1, 128)),
            full((1, 128)),
        ],
        out_specs=[
            pl.BlockSpec((EB, 16), lambda i: (i, 0)),
            pl.BlockSpec((EB, 128), lambda i: (i, 0)),
            pl.BlockSpec((EB, 128), lambda i: (i, 0)),
        ],
        out_shape=[
            jax.ShapeDtypeStruct((e, 16), jnp.float32),
            jax.ShapeDtypeStruct((e, 128), jnp.float32),
            jax.ShapeDtypeStruct((e, 128), jnp.float32),
        ],
        interpret=_INTERPRET,
    )(z, srg, pd, waT, w2aT, b2a, w3aT, b3a, wvT, b1v, w2vT, b2v,
      w3vT, b3v, seqb)
    return outs


def _update_body(last, num_ref, s_ref, woutT_ref, bout_ref,
                 wf1T_ref, bf1_ref, wf2T_ref, bf2_ref, wnT_ref, bn_ref,
                 s_out_ref, p_out_ref):
    inv = 1.0 / jnp.sqrt(1.0 + 1e-5)
    ao = jnp.concatenate(
        [(num_ref[0, h, :, 0:128] + num_ref[1, h, :, 0:128])
         / (num_ref[0, 4, :, h:h + 1] + num_ref[1, 4, :, h:h + 1] + 1e-12)
         for h in range(4)], axis=1)
    s = s_ref[...]
    s = s + (jnp.dot(ao, woutT_ref[...], preferred_element_type=jnp.float32)
             + bout_ref[...]) * inv
    t = _gelu(jnp.dot(s, wf1T_ref[...], preferred_element_type=jnp.float32)
              + bf1_ref[...])
    s = s + (jnp.dot(t, wf2T_ref[...], preferred_element_type=jnp.float32)
             + bf2_ref[...]) * inv
    s_out_ref[...] = s
    p = jnp.dot(s, wnT_ref[...], preferred_element_type=jnp.float32) + bn_ref[...]
    p_out_ref[...] = p
    del last


def _update_kernel(num, s, woutT, bout, wf1T, bf1, wf2T, bf2, wnT, bn):
    n = s.shape[0]
    pc = wnT.shape[1]
    grid = (n // NB,)
    full = lambda shape: pl.BlockSpec(shape, lambda i: tuple(0 for _ in shape))
    s_new, p_new = pl.pallas_call(
        functools.partial(_update_body, False),
        grid=grid,
        in_specs=[
            pl.BlockSpec((2, 5, NB, 128), lambda i: (0, 0, i, 0)),
            pl.BlockSpec((NB, 128), lambda i: (i, 0)),
            full((512, 128)), full((1, 128)),
            full((128, 128)), full((1, 128)),
            full((128, 128)), full((1, 128)),
            full((128, pc)), full((1, pc)),
        ],
        out_specs=[
            pl.BlockSpec((NB, 128), lambda i: (i, 0)),
            pl.BlockSpec((NB, pc), lambda i: (i, 0)),
        ],
        out_shape=[
            jax.ShapeDtypeStruct((n, 128), jnp.float32),
            jax.ShapeDtypeStruct((n, pc), jnp.float32),
        ],
        interpret=_INTERPRET,
    )(num, s, woutT, bout, wf1T, bf1, wf2T, bf2, wnT, bn)
    return s_new, p_new


def _gather_rows(tab, idx):
    return tab[idx]


def _vis_compute(rand, src, dst):
    return (rand[src] < rand[dst]).astype(jnp.float32)


def _aggregate(expaw, av, dst, n):
    e = expaw.shape[0]
    half = e // 2
    out = []
    for p in range(2):
        sl = slice(p * half, (p + 1) * half)
        vals = jnp.concatenate(
            [av[sl], jnp.ones((half, 1), jnp.float32),
             jnp.zeros((half, 15), jnp.float32)], axis=1)
        per_h = []
        for h in range(4):
            per_h.append(jax.ops.segment_sum(
                expaw[sl, h:h + 1] * vals, dst[sl], num_segments=n))
        out.append(jnp.stack(per_h, axis=0))
    return jnp.stack(out, axis=0)


def kernel(s, z, edge_idx, valid_mask, res_type_clone, params):
    n, d = s.shape
    kk = res_type_clone.shape[-1]
    src = edge_idx[0].astype(jnp.int32)
    dst = edge_idx[1].astype(jnp.int32)
    rand = jax.random.uniform(jax.random.key(42), (n,), dtype=s.dtype)
    rtc = (res_type_clone != 0).reshape(-1, kk).astype(s.dtype)

    seqW, seqb = params["seq_to_s"]
    layers = params["layers"]

    def lt(p):
        W, b = p
        return W.T, b[None, :]

    lws = []
    for lp in layers:
        w1a, b1a = lp["aw"][0]
        w2aT, b2a = lt(lp["aw"][1])
        w3aT, b3a = lt(lp["aw"][2])
        wvT, b1v = lt(lp["av"][0])
        w2vT, b2v = lt(lp["av"][1])
        w3vT, b3v = lt(lp["av"][2])
        wdT = w1a[:, 0:128].T
        waT = w1a[:, 128:384].T
        lws.append(dict(wdT=wdT, b1a=b1a[None, :],
                        ew=(waT, w2aT, b2a, w3aT, b3a,
                            wvT, b1v, w2vT, b2v, w3vT, b3v)))

    tab, p = _node_pre(s, rtc, rand, seqW.T, lws[0]["wdT"], lws[0]["b1a"])

    srg0 = _sc_gather(tab, src)
    nch = z.shape[0] // NW // CH
    nchp = ((nch + 7) // 8) * 8
    dst3d = jnp.pad(dst.reshape(NW, nch, CH),
                    ((0, 0), (0, nchp - nch), (0, 0)))

    u = None
    cur_s = s
    for li, lp in enumerate(layers):
        pd = _sc_gather(p, dst)
        if li == 0:
            expaw, av, u = _edge_kernel(True, z, srg0, pd,
                                        lws[li]["ew"], seqb[None, :])
        else:
            expaw, av, _ = _edge_kernel(False, z, u, pd,
                                        lws[li]["ew"], seqb[None, :])
        num = _sc_aggregate(expaw, av, dst3d, n)
        woutT, bout = lt(layers[li]["out"])
        wf1T, bf1 = lt(layers[li]["ffn"][0])
        wf2T, bf2 = lt(layers[li]["ffn"][1])
        if li + 1 < len(layers):
            wnT, bn = lws[li + 1]["wdT"], lws[li + 1]["b1a"]
        else:
            predW = params["pred_W"]
            wnT, bn = predW.T, jnp.zeros((1, predW.shape[0]), jnp.float32)
        cur_s, p = _update_kernel(num, cur_s, woutT, bout, wf1T, bf1,
                                  wf2T, bf2, wnT, bn)

    logits = p
    bm, nn = valid_mask.shape
    return logits.reshape(bm, nn, kk)


# speedup vs baseline: 2.7532x; 1.3218x over previous
import functools

import jax
import jax.numpy as jnp
from jax import lax
from jax.experimental import pallas as pl
from jax.experimental.pallas import tpu as pltpu
from jax.experimental.pallas import tpu_sc as plsc

_INTERPRET = False

EB = 512
NB = 1000
NW = 32
CH = 40


def _sc_mesh():
    return plsc.VectorSubcoreMesh(core_axis_name="c", subcore_axis_name="s")


def _sc_gather(tab, idx):
    e = idx.shape[0]
    cols = tab.shape[1]
    per_w = e // NW
    nch = per_w // CH

    @functools.partial(
        pl.kernel, mesh=_sc_mesh(),
        out_type=jax.ShapeDtypeStruct((e, cols), jnp.float32),
        scratch_types=[
            pltpu.VMEM((per_w,), jnp.int32),
            pltpu.VMEM((CH, cols), jnp.float32),
            pltpu.SemaphoreType.DMA,
        ],
    )
    def k(tab_hbm, idx_hbm, out_hbm, idx_v, rows_v, sem):
        wid = lax.axis_index("c") * 16 + lax.axis_index("s")
        base = wid * per_w
        pltpu.sync_copy(idx_hbm.at[pl.ds(base, per_w)], idx_v)

        def body(j, _):
            off = pl.multiple_of(j * CH, CH)
            pltpu.async_copy(tab_hbm.at[idx_v.at[pl.ds(off, CH)]],
                             rows_v, sem).wait()
            pltpu.sync_copy(rows_v, out_hbm.at[pl.ds(base + off, CH)])
            return 0

        lax.fori_loop(0, nch, body, 0)

    return k(tab, idx)


def _sc_aggregate(expaw, av, dst3d, n):
    e = av.shape[0]
    per_w = e // NW
    nch = per_w // CH
    nchp = dst3d.shape[1]
    npad = ((n + 2047) // 2048) * 2048
    rpt = npad // 16
    qr = CH
    nq = rpt // qr

    @functools.partial(
        pl.kernel, mesh=_sc_mesh(),
        out_type=jax.ShapeDtypeStruct((2, 5, npad, 128), jnp.float32),
        scratch_types=[
            pltpu.VMEM((8, CH), jnp.int32),
            pltpu.VMEM((CH, 16), jnp.float32),
            pltpu.VMEM((CH, 16), jnp.float32),
            pltpu.VMEM((CH, 128), jnp.float32),
            pltpu.VMEM((CH, 128), jnp.float32),
            pltpu.VMEM((CH, 128), jnp.float32),
            pltpu.VMEM_SHARED((npad, 128), jnp.float32),
            pltpu.SemaphoreType.DMA,
            pltpu.SemaphoreType.DMA,
            pltpu.SemaphoreType.DMA,
            pltpu.SemaphoreType.DMA,
            pltpu.SemaphoreType.DMA,
        ],
    )
    def k(ea_hbm, av_hbm, dst_hbm, out_hbm,
          idx_v, ea0, ea1, av0, av1, vl0, table,
          sle0, sle1, slv0, slv1, ss):
        ea_b = (ea0, ea1)
        av_b = (av0, av1)
        sle = (sle0, sle1)
        slv = (slv0, slv1)
        sc = lax.axis_index("c")
        tid = lax.axis_index("s")
        wid = sc * 16 + tid
        base = pl.multiple_of(wid * per_w, 8)
        row0 = pl.multiple_of(tid * rpt, 8)
        nblk = nchp // 8

        def eslice(j):
            return pl.ds(pl.multiple_of(base + j * CH, 8), CH)

        for h in range(5):
            def zrow(q, _):
                for c in range(8):
                    av0[q, pl.ds(c * 16, 16)] = jnp.zeros((16,), jnp.float32)
                return 0
            lax.fori_loop(0, qr, zrow, 0)
            for q in range(nq):
                pltpu.sync_copy(
                    av0,
                    table.at[pl.ds(pl.multiple_of(row0 + q * qr, 8), qr)])
            if h == 4:
                def zv(ei, _):
                    for c in range(1, 8):
                        vl0[ei, pl.ds(c * 16, 16)] = jnp.zeros(
                            (16,), jnp.float32)
                    return 0
                lax.fori_loop(0, CH, zv, 0)
            plsc.subcore_barrier()

            for b in range(2):
                pltpu.async_copy(ea_hbm.at[eslice(b)], ea_b[b], sle[b])
                if h < 4:
                    pltpu.async_copy(av_hbm.at[eslice(b)], av_b[b], slv[b])

            def blk(jo, _):
                @pl.when(jo > 0)
                def _():
                    pltpu.make_async_copy(vl0, table.at[idx_v.at[0]],
                                          ss).wait()
                pltpu.sync_copy(
                    dst_hbm.at[wid, pl.ds(pl.multiple_of(jo * 8, 8), 8)],
                    idx_v)
                for b2 in range(8):
                    j = jo * 8 + b2
                    b = b2 % 2

                    @pl.when(j < nch)
                    def _():
                        pltpu.make_async_copy(ea_hbm.at[eslice(j)], ea_b[b],
                                              sle[b]).wait()
                        if h < 4:
                            pltpu.make_async_copy(av_hbm.at[eslice(j)],
                                                  av_b[b], slv[b]).wait()

                        if b2 > 0:
                            pltpu.make_async_copy(
                                vl0, table.at[idx_v.at[b2]], ss).wait()

                        if h < 4:
                            def body(ei, _):
                                evec = ea_b[b][ei, pl.ds(0, 16)]
                                scv = evec[h]
                                for c in range(8):
                                    vl0[ei, pl.ds(c * 16, 16)] = (
                                        scv * av_b[b][ei, pl.ds(c * 16, 16)])
                                return 0
                        else:
                            def body(ei, _):
                                vl0[ei, pl.ds(0, 16)] = ea_b[b][
                                    ei, pl.ds(0, 16)]
                                return 0
                        lax.fori_loop(0, CH, body, 0)
                        pltpu.async_copy(vl0, table.at[idx_v.at[b2]], ss,
                                         add=True)

                        @pl.when(j + 2 < nch)
                        def _():
                            pltpu.async_copy(ea_hbm.at[eslice(j + 2)],
                                             ea_b[b], sle[b])
                            if h < 4:
                                pltpu.async_copy(av_hbm.at[eslice(j + 2)],
                                                 av_b[b], slv[b])
                return 0
            lax.fori_loop(0, nblk, blk, 0)
            pltpu.make_async_copy(vl0, table.at[idx_v.at[0]], ss).wait()
            plsc.subcore_barrier()

            for q in range(nq):
                roff = pl.multiple_of(row0 + q * qr, 8)
                pltpu.sync_copy(table.at[pl.ds(roff, qr)], vl0)
                pltpu.sync_copy(vl0, out_hbm.at[sc, h, pl.ds(roff, qr)])
            plsc.subcore_barrier()

    return k(expaw, av, dst3d)


def _gelu(x):
    return x * 0.5 * (1.0 + jax.lax.erf(x / jnp.sqrt(2.0).astype(x.dtype)))


def _node_pre_body(s0_ref, rtc_ref, rand_ref, seqWT_ref, wd0T_ref, b1a0_ref,
                   tab_ref, p0_ref):
    s0 = s0_ref[...]
    rand = rand_ref[...]
    r = jnp.dot(rtc_ref[...], seqWT_ref[...],
                preferred_element_type=jnp.float32)
    tab_ref[:, 0:128] = s0
    tab_ref[:, 128:256] = r
    tab_ref[:, 256:257] = rand
    tab_ref[:, 257:384] = jnp.zeros_like(tab_ref[:, 257:384])
    p0_ref[:, 0:128] = jnp.dot(s0, wd0T_ref[...],
                               preferred_element_type=jnp.float32) + b1a0_ref[...]
    p0_ref[:, 128:129] = rand
    p0_ref[:, 129:256] = jnp.zeros_like(p0_ref[:, 129:256])


def _node_pre(s0, rtc, rand, seqWT, wd0T, b1a0):
    n = s0.shape[0]
    grid = (n // NB,)
    tab, p0 = pl.pallas_call(
        _node_pre_body,
        grid=grid,
        in_specs=[
            pl.BlockSpec((NB, 128), lambda i: (i, 0)),
            pl.BlockSpec((NB, 33), lambda i: (i, 0)),
            pl.BlockSpec((NB, 1), lambda i: (i, 0)),
            pl.BlockSpec((33, 128), lambda i: (0, 0)),
            pl.BlockSpec((128, 128), lambda i: (0, 0)),
            pl.BlockSpec((1, 128), lambda i: (0, 0)),
        ],
        out_specs=[
            pl.BlockSpec((NB, 384), lambda i: (i, 0)),
            pl.BlockSpec((NB, 256), lambda i: (i, 0)),
        ],
        out_shape=[
            jax.ShapeDtypeStruct((n, 384), jnp.float32),
            jax.ShapeDtypeStruct((n, 256), jnp.float32),
        ],
        interpret=_INTERPRET,
    )(s0, rtc, rand[:, None], seqWT, wd0T, b1a0)
    return tab, p0


def _edge_body(first, z_ref, srg_ref, pd_ref,
               waT_ref, w2aT_ref, b2a_ref, w3aT_ref, b3a_ref,
               wvT_ref, b1v_ref, w2vT_ref, b2v_ref, w3vT_ref, b3v_ref,
               seqb_ref,
               expaw_ref, av_ref, u_ref):
    z = z_ref[...]
    if first:
        vis = jnp.where(srg_ref[:, 256:257] < pd_ref[:, 128:129], 1.0, 0.0)
        u = (srg_ref[:, 0:128]
             + vis * srg_ref[:, 128:256]
             + seqb_ref[...])
        u_ref[...] = u
    else:
        u = srg_ref[...]
    zu = jnp.concatenate([z, u], axis=1)
    h = (jnp.dot(zu, waT_ref[...], preferred_element_type=jnp.float32)
         + pd_ref[:, 0:128])
    h = _gelu(h)
    h = _gelu(jnp.dot(h, w2aT_ref[...], preferred_element_type=jnp.float32)
              + b2a_ref[...])
    aw = jnp.dot(h, w3aT_ref[...], preferred_element_type=jnp.float32) + b3a_ref[...]
    expaw_ref[:, 0:4] = jnp.exp(aw)
    expaw_ref[:, 4:16] = jnp.zeros_like(expaw_ref[:, 4:16])
    g = _gelu(jnp.dot(zu, wvT_ref[...], preferred_element_type=jnp.float32)
              + b1v_ref[...])
    g = _gelu(jnp.dot(g, w2vT_ref[...], preferred_element_type=jnp.float32)
              + b2v_ref[...])
    av_ref[...] = (jnp.dot(g, w3vT_ref[...], preferred_element_type=jnp.float32)
                   + b3v_ref[...])


def _edge_kernel(first, z, srg, pd, lw, seqb):
    e = z.shape[0]
    grid = (e // EB,)
    (waT, w2aT, b2a, w3aT, b3a, wvT, b1v, w2vT, b2v, w3vT, b3v) = lw
    full = lambda shape: pl.BlockSpec(shape, lambda i: tuple(0 for _ in shape))
    srg_cols = srg.shape[1]
    pd_cols = pd.shape[1]
    outs = pl.pallas_call(
        functools.partial(_edge_body, first),
        grid=grid,
        in_specs=[
            pl.BlockSpec((EB, 128), lambda i: (i, 0)),
            pl.BlockSpec((EB, srg_cols), lambda i: (i, 0)),
            pl.BlockSpec((EB, pd_cols), lambda i: (i, 0)),
            full((256, 128)), full((128, 128)), full((1, 128)),
            full((128, 4)), full((1, 4)),
            full((256, 128)), full((1, 128)), full((128, 128)), full((1, 128)),
            full((128, 128)), full((1, 128)),
            full((1, 128)),
        ],
        out_specs=[
            pl.BlockSpec((EB, 16), lambda i: (i, 0)),
            pl.BlockSpec((EB, 128), lambda i: (i, 0)),
            pl.BlockSpec((EB, 128), lambda i: (i, 0)),
        ],
        out_shape=[
            jax.ShapeDtypeStruct((e, 16), jnp.float32),
            jax.ShapeDtypeStruct((e, 128), jnp.float32),
            jax.ShapeDtypeStruct((e, 128), jnp.float32),
        ],
        interpret=_INTERPRET,
    )(z, srg, pd, waT, w2aT, b2a, w3aT, b3a, wvT, b1v, w2vT, b2v,
      w3vT, b3v, seqb)
    return outs


def _update_body(last, num_ref, s_ref, woutT_ref, bout_ref,
                 wf1T_ref, bf1_ref, wf2T_ref, bf2_ref, wnT_ref, bn_ref,
                 s_out_ref, p_out_ref):
    inv = 1.0 / jnp.sqrt(1.0 + 1e-5)
    ao = jnp.concatenate(
        [(num_ref[0, h, :, 0:128] + num_ref[1, h, :, 0:128])
         / (num_ref[0, 4, :, h:h + 1] + num_ref[1, 4, :, h:h + 1] + 1e-12)
         for h in range(4)], axis=1)
    s = s_ref[...]
    s = s + (jnp.dot(ao, woutT_ref[...], preferred_element_type=jnp.float32)
             + bout_ref[...]) * inv
    t = _gelu(jnp.dot(s, wf1T_ref[...], preferred_element_type=jnp.float32)
              + bf1_ref[...])
    s = s + (jnp.dot(t, wf2T_ref[...], preferred_element_type=jnp.float32)
             + bf2_ref[...]) * inv
    s_out_ref[...] = s
    p = jnp.dot(s, wnT_ref[...], preferred_element_type=jnp.float32) + bn_ref[...]
    p_out_ref[...] = p
    del last


def _update_kernel(num, s, woutT, bout, wf1T, bf1, wf2T, bf2, wnT, bn):
    n = s.shape[0]
    pc = wnT.shape[1]
    grid = (n // NB,)
    full = lambda shape: pl.BlockSpec(shape, lambda i: tuple(0 for _ in shape))
    s_new, p_new = pl.pallas_call(
        functools.partial(_update_body, False),
        grid=grid,
        in_specs=[
            pl.BlockSpec((2, 5, NB, 128), lambda i: (0, 0, i, 0)),
            pl.BlockSpec((NB, 128), lambda i: (i, 0)),
            full((512, 128)), full((1, 128)),
            full((128, 128)), full((1, 128)),
            full((128, 128)), full((1, 128)),
            full((128, pc)), full((1, pc)),
        ],
        out_specs=[
            pl.BlockSpec((NB, 128), lambda i: (i, 0)),
            pl.BlockSpec((NB, pc), lambda i: (i, 0)),
        ],
        out_shape=[
            jax.ShapeDtypeStruct((n, 128), jnp.float32),
            jax.ShapeDtypeStruct((n, pc), jnp.float32),
        ],
        interpret=_INTERPRET,
    )(num, s, woutT, bout, wf1T, bf1, wf2T, bf2, wnT, bn)
    return s_new, p_new


def _gather_rows(tab, idx):
    return tab[idx]


def _vis_compute(rand, src, dst):
    return (rand[src] < rand[dst]).astype(jnp.float32)


def _aggregate(expaw, av, dst, n):
    e = expaw.shape[0]
    half = e // 2
    out = []
    for p in range(2):
        sl = slice(p * half, (p + 1) * half)
        vals = jnp.concatenate(
            [av[sl], jnp.ones((half, 1), jnp.float32),
             jnp.zeros((half, 15), jnp.float32)], axis=1)
        per_h = []
        for h in range(4):
            per_h.append(jax.ops.segment_sum(
                expaw[sl, h:h + 1] * vals, dst[sl], num_segments=n))
        out.append(jnp.stack(per_h, axis=0))
    return jnp.stack(out, axis=0)


def kernel(s, z, edge_idx, valid_mask, res_type_clone, params):
    n, d = s.shape
    kk = res_type_clone.shape[-1]
    src = edge_idx[0].astype(jnp.int32)
    dst = edge_idx[1].astype(jnp.int32)
    rand = jax.random.uniform(jax.random.key(42), (n,), dtype=s.dtype)
    rtc = (res_type_clone != 0).reshape(-1, kk).astype(s.dtype)

    seqW, seqb = params["seq_to_s"]
    layers = params["layers"]

    def lt(p):
        W, b = p
        return W.T, b[None, :]

    lws = []
    for lp in layers:
        w1a, b1a = lp["aw"][0]
        w2aT, b2a = lt(lp["aw"][1])
        w3aT, b3a = lt(lp["aw"][2])
        wvT, b1v = lt(lp["av"][0])
        w2vT, b2v = lt(lp["av"][1])
        w3vT, b3v = lt(lp["av"][2])
        wdT = w1a[:, 0:128].T
        waT = w1a[:, 128:384].T
        lws.append(dict(wdT=wdT, b1a=b1a[None, :],
                        ew=(waT, w2aT, b2a, w3aT, b3a,
                            wvT, b1v, w2vT, b2v, w3vT, b3v)))

    tab, p = _node_pre(s, rtc, rand, seqW.T, lws[0]["wdT"], lws[0]["b1a"])

    srg0 = _sc_gather(tab, src)
    nch = z.shape[0] // NW // CH
    nchp = ((nch + 7) // 8) * 8
    dst3d = jnp.pad(dst.reshape(NW, nch, CH),
                    ((0, 0), (0, nchp - nch), (0, 0)))

    u = None
    cur_s = s
    for li, lp in enumerate(layers):
        pd = _sc_gather(p, dst)
        if li == 0:
            expaw, av, u = _edge_kernel(True, z, srg0, pd,
                                        lws[li]["ew"], seqb[None, :])
        else:
            expaw, av, _ = _edge_kernel(False, z, u, pd,
                                        lws[li]["ew"], seqb[None, :])
        num = _sc_aggregate(expaw, av, dst3d, n)
        woutT, bout = lt(layers[li]["out"])
        wf1T, bf1 = lt(layers[li]["ffn"][0])
        wf2T, bf2 = lt(layers[li]["ffn"][1])
        if li + 1 < len(layers):
            wnT, bn = lws[li + 1]["wdT"], lws[li + 1]["b1a"]
        else:
            predW = params["pred_W"]
            wnT, bn = predW.T, jnp.zeros((1, predW.shape[0]), jnp.float32)
        cur_s, p = _update_kernel(num, cur_s, woutT, bout, wf1T, bf1,
                                  wf2T, bf2, wnT, bn)

    logits = p
    bm, nn = valid_mask.shape
    return logits.reshape(bm, nn, kk)
